# Initial kernel scaffold; baseline (speedup 1.0000x reference)
import jax
import jax.numpy as jnp
from jax.experimental import pallas as pl


def kernel(pos, x0, x1, x2, cell1, cell2, send_0_0, recv_0_0, send_0_1, recv_0_1, send_1_1, recv_1_1, send_1_2, recv_1_2, params):
    raise NotImplementedError("write your pallas kernel here")


# speedup vs baseline: 3.1975x; 3.1975x over previous
import functools

import jax
import jax.numpy as jnp
from jax import lax
from jax.experimental import pallas as pl
from jax.experimental.pallas import tpu as pltpu
from jax.experimental.pallas import tpu_sc as plsc

F32 = jnp.float32
I32 = jnp.int32

NC, NS, CH = 2, 16, 128
NW = NC * NS
TILE = NW * CH

N0, N1, N2 = 10000, 20000, 5000
H = 64

ADJ = [('0_0', '0', '0'), ('0_1', '0', '1'), ('1_1', '1', '1'), ('1_2', '1', '2')]
ECNT = [320000, 40000, 60000, 30000]
NCELLS = {'0': N0, '1': N1, '2': N2}


def _rup(x, m):
    return (x + m - 1) // m * m


EPAD = [_rup(e, TILE) for e in ECNT]
EOFF = [sum(EPAD[:i]) for i in range(4)]
ETOT = sum(EPAD)
KG = ETOT // TILE
KA = [e // TILE for e in EPAD]
NT = [NCELLS[t] for _, _, t in ADJ]
NTPAD = [_rup(n + 1, NS * 8) for n in NT]

ROFF = {'0': 0, '1': N0, '2': N0 + N1}
NROWS = N0 + N1 + N2

TREG_SRC = []
TREG_OFF = []
_off = 0
for _a, _s, _t in ADJ:
    for _r in (_s, _t):
        TREG_SRC.append(_r)
        TREG_OFF.append(_off)
        _off += NCELLS[_r]
TROWS = _off

CELLROWS = N1 * 2 + N2 * 6
CELLPAD = _rup(CELLROWS, TILE)
KC = CELLPAD // TILE

EBLK = 2048
EBND = [sum(EPAD[:i + 1]) // EBLK for i in range(4)]

TBLK = 1000
_treg_nblk = [NCELLS[r] // TBLK for r in TREG_SRC]
TBND = [sum(_treg_nblk[:i + 1]) for i in range(8)]
TSRC_BLK = [ROFF[r] // TBLK for r in TREG_SRC]


def _piecewise(i, bounds, vals):
    v = jnp.int32(vals[0])
    for b, nxt in zip(bounds, vals[1:]):
        v = jnp.where(i >= b, jnp.int32(nxt), v)
    return v


def _sc_mesh():
    return plsc.VectorSubcoreMesh(core_axis_name="c", subcore_axis_name="s",
                                  num_cores=NC, num_subcores=NS)


@functools.lru_cache(maxsize=None)
def _make_gather(nrows_out, k, d):
    @functools.partial(
        pl.kernel,
        out_type=jax.ShapeDtypeStruct((nrows_out, d), F32),
        mesh=_sc_mesh(),
        scratch_types=[pltpu.VMEM((k, CH), I32), pltpu.VMEM((CH, d), F32)],
        compiler_params=pltpu.CompilerParams(use_tc_tiling_on_sc=False),
    )
    def gather_k(tab_hbm, idx_hbm, out_hbm, vidx, vrow):
        c = lax.axis_index("c")
        s = lax.axis_index("s")
        w = c * NS + s
        pltpu.sync_copy(idx_hbm.at[w], vidx)

        def body(j, carry):
            pltpu.sync_copy(tab_hbm.at[vidx.at[j]], vrow)
            pltpu.sync_copy(vrow, out_hbm.at[pl.ds((w * k + j) * CH, CH)])
            return carry

        lax.fori_loop(0, k, body, 0)

    return gather_k


@functools.lru_cache(maxsize=None)
def _make_scatter(k, ntpad, eoff):
    rpt = ntpad // NS

    @functools.partial(
        pl.kernel,
        out_type=jax.ShapeDtypeStruct((NC * ntpad, H), F32),
        mesh=_sc_mesh(),
        scratch_types=[pltpu.VMEM_SHARED((ntpad, H), F32),
                       pltpu.VMEM((k, CH), I32),
                       pltpu.VMEM((CH, H), F32)],
        compiler_params=pltpu.CompilerParams(use_tc_tiling_on_sc=False),
    )
    def scatter_k(m_hbm, idx_hbm, zeros_hbm, out_hbm, acc, vidx, vm):
        c = lax.axis_index("c")
        s = lax.axis_index("s")
        w = c * NS + s
        pltpu.sync_copy(idx_hbm.at[w], vidx)
        pltpu.sync_copy(zeros_hbm.at[pl.ds(0, rpt)], acc.at[pl.ds(s * rpt, rpt)])
        plsc.subcore_barrier()

        def body(j, carry):
            pltpu.sync_copy(m_hbm.at[pl.ds(eoff + (w * k + j) * CH, CH)], vm)
            pltpu.sync_copy(vm, acc.at[vidx.at[j]], add=True)
            return carry

        lax.fori_loop(0, k, body, 0)
        plsc.subcore_barrier()
        pltpu.sync_copy(acc.at[pl.ds(s * rpt, rpt)],
                        out_hbm.at[pl.ds((c * NS + s) * rpt, rpt)])

    return scatter_k


def _linear_body(x_ref, w_ref, b_ref, o_ref):
    o_ref[...] = (jnp.dot(x_ref[...], w_ref[...], preferred_element_type=F32)
                  + b_ref[...])


def _linear(x, w, b, bn=1000):
    n, din = x.shape
    dout = w.shape[1]
    return pl.pallas_call(
        _linear_body,
        grid=(n // bn,),
        in_specs=[pl.BlockSpec((bn, din), lambda i: (i, 0)),
                  pl.BlockSpec((din, dout), lambda i: (0, 0)),
                  pl.BlockSpec((1, dout), lambda i: (0, 0))],
        out_specs=pl.BlockSpec((bn, dout), lambda i: (i, 0)),
        out_shape=jax.ShapeDtypeStruct((n, dout), F32),
    )(x, w, b.reshape(1, dout))


def _geom(pg, kpts, bn=1000):
    n = pg.shape[0]

    def body(p_ref, o_ref):
        pts = [p_ref[:, i, :] for i in range(kpts)]
        c = pts[0]
        for p in pts[1:]:
            c = c + p
        c = c * (1.0 / kpts)
        col = lax.broadcasted_iota(I32, (bn, 16), 1)
        m3 = col < 3
        d2 = None
        for p in pts:
            diff = p - c
            dd = jnp.sum(jnp.where(m3, diff * diff, 0.0), axis=1)
            d2 = dd if d2 is None else jnp.maximum(d2, dd)
        diam = jnp.sqrt(d2)
        o_ref[...] = (jnp.where(m3, c, 0.0)
                      + jnp.where(col == 3, diam[:, None], 0.0))

    return pl.pallas_call(
        body,
        grid=(n // bn,),
        in_specs=[pl.BlockSpec((bn, kpts, 16), lambda i: (i, 0, 0))],
        out_specs=pl.BlockSpec((bn, 16), lambda i: (i, 0)),
        out_shape=jax.ShapeDtypeStruct((n, 16), F32),
    )(pg)


def _edge_dist(ges, get):
    def body(s_ref, t_ref, o_ref):
        diff = s_ref[...] - t_ref[...]
        col16 = lax.broadcasted_iota(I32, (EBLK, 16), 1)
        d2 = jnp.sum(jnp.where(col16 < 3, diff * diff, 0.0), axis=1)
        d = jnp.sqrt(d2)
        col8 = lax.broadcasted_iota(I32, (EBLK, 8), 1)
        o_ref[...] = jnp.where(col8 == 0, d[:, None], 0.0)

    return pl.pallas_call(
        body,
        grid=(ETOT // EBLK,),
        in_specs=[pl.BlockSpec((EBLK, 16), lambda i: (i, 0)),
                  pl.BlockSpec((EBLK, 16), lambda i: (i, 0))],
        out_specs=pl.BlockSpec((EBLK, 8), lambda i: (i, 0)),
        out_shape=jax.ShapeDtypeStruct((ETOT, 8), F32),
    )(ges, get)


def _build_table(xcat, diamc, wstk, auxstk):
    def rid(i):
        return _piecewise(i, TBND[:-1], list(range(8)))

    def src(i):
        off = _piecewise(i, TBND[:-1],
                         [TSRC_BLK[r] - ([0] + TBND)[r] for r in range(8)])
        return i + off

    def body(x_ref, d_ref, w_ref, a_ref, o_ref):
        a = a_ref[...]
        y = jnp.dot(x_ref[...], w_ref[...][0], preferred_element_type=F32)
        o_ref[...] = y + d_ref[...] * a[0, 0][None, :] + a[0, 1][None, :]

    return pl.pallas_call(
        body,
        grid=(TBND[-1],),
        in_specs=[pl.BlockSpec((TBLK, H), lambda i: (src(i), 0)),
                  pl.BlockSpec((TBLK, 1), lambda i: (src(i), 0)),
                  pl.BlockSpec((1, H, H), lambda i: (rid(i), 0, 0)),
                  pl.BlockSpec((1, 2, H), lambda i: (rid(i), 0, 0))],
        out_specs=pl.BlockSpec((TBLK, H), lambda i: (i, 0)),
        out_shape=jax.ShapeDtypeStruct((TROWS, H), F32),
    )(xcat, diamc, wstk, auxstk)


def _edge_mlp(us, ut, d8, wd, w2, b2):
    def aid(i):
        return _piecewise(i, EBND[:-1], list(range(4)))

    def body(s_ref, t_ref, d_ref, wd_ref, w2_ref, b2_ref, o_ref):
        pre = (s_ref[...] + t_ref[...]
               + d_ref[:, 0:1] * wd_ref[...][0, 0][None, :])
        h = jax.nn.silu(pre)
        m = (jnp.dot(h, w2_ref[...][0], preferred_element_type=F32)
             + b2_ref[...][0, 0][None, :])
        o_ref[...] = jax.nn.silu(m)

    return pl.pallas_call(
        body,
        grid=(EBND[-1],),
        in_specs=[pl.BlockSpec((EBLK, H), lambda i: (i, 0)),
                  pl.BlockSpec((EBLK, H), lambda i: (i, 0)),
                  pl.BlockSpec((EBLK, 8), lambda i: (i, 0)),
                  pl.BlockSpec((1, 1, H), lambda i: (aid(i), 0, 0)),
                  pl.BlockSpec((1, H, H), lambda i: (aid(i), 0, 0)),
                  pl.BlockSpec((1, 1, H), lambda i: (aid(i), 0, 0))],
        out_specs=pl.BlockSpec((EBLK, H), lambda i: (i, 0)),
        out_shape=jax.ShapeDtypeStruct((ETOT, H), F32),
    )(us, ut, d8, wd, w2, b2)


def _update(xr, aggs, w, b, bn=1000):
    n = xr.shape[0]
    nagg = len(aggs)

    def body(x_ref, *rest):
        agg_refs = rest[:nagg]
        w_ref, b_ref, o_ref = rest[nagg], rest[nagg + 1], rest[nagg + 2]
        x = x_ref[...]
        w_full = w_ref[...]
        y = x + jnp.dot(x, w_full[0:H], preferred_element_type=F32) + b_ref[...]
        for i, a_ref in enumerate(agg_refs):
            a = a_ref[...]
            asum = a[0] + a[1]
            y = y + jnp.dot(asum, w_full[(i + 1) * H:(i + 2) * H],
                            preferred_element_type=F32)
        o_ref[...] = y

    in_specs = [pl.BlockSpec((bn, H), lambda i: (i, 0))]
    for _ in aggs:
        in_specs.append(pl.BlockSpec((NC, bn, H), lambda i: (0, i, 0)))
    in_specs.append(pl.BlockSpec(((1 + nagg) * H, H), lambda i: (0, 0)))
    in_specs.append(pl.BlockSpec((1, H), lambda i: (0, 0)))
    return pl.pallas_call(
        body,
        grid=(n // bn,),
        in_specs=in_specs,
        out_specs=pl.BlockSpec((bn, H), lambda i: (i, 0)),
        out_shape=jax.ShapeDtypeStruct((n, H), F32),
    )(xr, *aggs, w, b.reshape(1, H))


def _pad_to(a, n, value=0):
    return jnp.concatenate([a, jnp.full((n - a.shape[0],), value, a.dtype)])


def kernel(pos, x0, x1, x2, cell1, cell2, send_0_0, recv_0_0, send_0_1,
           recv_0_1, send_1_1, recv_1_1, send_1_2, recv_1_2, params):
    snd = {'0_0': send_0_0, '0_1': send_0_1, '1_1': send_1_1, '1_2': send_1_2}
    rcv = {'0_0': recv_0_0, '0_1': recv_0_1, '1_1': recv_1_1, '1_2': recv_1_2}

    cell_idx = _pad_to(jnp.concatenate([cell1.reshape(-1), cell2.reshape(-1)]),
                       CELLPAD).astype(I32).reshape(NW, KC, CH)

    gidx, tidx_s, tidx_t = [], [], []
    for ai, (a, s, t) in enumerate(ADJ):
        sp = _pad_to(snd[a].astype(I32), EPAD[ai])
        tp = _pad_to(rcv[a].astype(I32), EPAD[ai])
        gidx.append((sp + ROFF[s], tp + ROFF[t]))
        tidx_s.append(sp + TREG_OFF[2 * ai])
        tidx_t.append(tp + TREG_OFF[2 * ai + 1])
    gidx_s = jnp.concatenate([g[0] for g in gidx]).reshape(NW, KG, CH)
    gidx_t = jnp.concatenate([g[1] for g in gidx]).reshape(NW, KG, CH)
    tidx_s = jnp.concatenate(tidx_s).reshape(NW, KG, CH)
    tidx_t = jnp.concatenate(tidx_t).reshape(NW, KG, CH)

    sidx = []
    for ai, (a, s, t) in enumerate(ADJ):
        rp = _pad_to(rcv[a].astype(I32), EPAD[ai], value=NT[ai])
        sidx.append(rp.reshape(NW, KA[ai], CH))

    zeros_hbm = jnp.zeros((max(NTPAD) // NS, H), F32)

    posp = jnp.pad(pos, ((0, 0), (0, 13)))
    pg = _make_gather(CELLPAD, KC, 16)(posp, cell_idx)
    g1 = _geom(pg[:N1 * 2].reshape(N1, 2, 16), 2)
    g2 = _geom(pg[N1 * 2:N1 * 2 + N2 * 6].reshape(N2, 6, 16), 6)
    gcat = jnp.concatenate([posp, g1, g2])
    diamc = gcat[:, 3:4]

    ges = _make_gather(ETOT, KG, 16)(gcat, gidx_s)
    get = _make_gather(ETOT, KG, 16)(gcat, gidx_t)
    d8 = _edge_dist(ges, get)

    emb = params['emb']
    xcat = jnp.concatenate([
        _linear(x0, emb['0'][0], emb['0'][1]),
        _linear(x1, emb['1'][0], emb['1'][1]),
        _linear(x2, emb['2'][0], emb['2'][1]),
    ])

    for layer in params['layers']:
        wstk, auxstk = [], []
        for a, _, _ in ADJ:
            w1 = layer['msg'][a]['W1']
            b1 = layer['msg'][a]['b1']
            wstk.append(w1[0:H])
            auxstk.append(jnp.stack([w1[2 * H + 1], jnp.zeros((H,), F32)]))
            wstk.append(w1[H:2 * H])
            auxstk.append(jnp.stack([w1[2 * H + 2], b1]))
        table = _build_table(xcat, diamc, jnp.stack(wstk), jnp.stack(auxstk))

        us = _make_gather(ETOT, KG, H)(table, tidx_s)
        ut = _make_gather(ETOT, KG, H)(table, tidx_t)

        wd = jnp.stack([layer['msg'][a]['W1'][2 * H][None, :] for a, _, _ in ADJ])
        w2 = jnp.stack([layer['msg'][a]['W2'] for a, _, _ in ADJ])
        b2 = jnp.stack([layer['msg'][a]['b2'][None, :] for a, _, _ in ADJ])
        m = _edge_mlp(us, ut, d8, wd, w2, b2)

        agg = {}
        for ai, (a, s, t) in enumerate(ADJ):
            part = _make_scatter(KA[ai], NTPAD[ai], EOFF[ai])(
                m, sidx[ai], zeros_hbm)
            agg[a] = part.reshape(NC, NTPAD[ai], H)

        upd = layer['upd']
        xcat = jnp.concatenate([
            _update(xcat[0:N0], [agg['0_0'][:, 0:N0]], upd['0'][0], upd['0'][1]),
            _update(xcat[N0:N0 + N1],
                    [agg['0_1'][:, 0:N1], agg['1_1'][:, 0:N1]],
                    upd['1'][0], upd['1'][1]),
            _update(xcat[N0 + N1:NROWS], [agg['1_2'][:, 0:N2]],
                    upd['2'][0], upd['2'][1]),
        ])

    pp = params['pre_pool']
    out0 = _linear(xcat[0:N0], pp['0'][0], pp['0'][1])
    out1 = _linear(xcat[N0:N0 + N1], pp['1'][0], pp['1'][1])
    out2 = _linear(xcat[N0 + N1:NROWS], pp['2'][0], pp['2'][1])
    return (out0, out1, out2, pos)
